# baseline (speedup 1.0000x reference)
import functools
import jax
import jax.numpy as jnp
from jax import lax
from jax.experimental import pallas as pl
from jax.experimental.pallas import tpu as pltpu

B, M, D, E = 2, 2048, 768, 8
T = B * M
TB1 = 1024
TB2 = 1024
NB1 = T // TB1
NB2 = T // TB2
HIGH = lax.Precision.HIGHEST


def _gating_body(x_ref, wg_ref, gates_ref, imp_ref, load_ref):
    logits = jnp.dot(x_ref[...].astype(jnp.bfloat16),
                     wg_ref[...].astype(jnp.bfloat16),
                     preferred_element_type=jnp.float32)
    eidx = lax.broadcasted_iota(jnp.int32, (TB1, E), 1)
    m1 = jnp.max(logits, axis=1, keepdims=True)
    a1 = jnp.min(jnp.where(logits == m1, eidx, E), axis=1, keepdims=True)
    masked = jnp.where(eidx == a1, -jnp.inf, logits)
    m2 = jnp.max(masked, axis=1, keepdims=True)
    a2 = jnp.min(jnp.where(masked == m2, eidx, E), axis=1, keepdims=True)
    ed = jnp.exp(m2 - m1)
    g1 = 1.0 / (1.0 + ed)
    g2 = ed / (1.0 + ed)
    gates = (jnp.where(eidx == a1, g1, 0.0)
             + jnp.where(eidx == a2, g2, 0.0))
    gates_ref[...] = gates
    imp_ref[0, 0, :] = jnp.sum(gates, axis=0)
    load_ref[0, 0, :] = jnp.sum((gates > 0.0).astype(jnp.float32), axis=0)


def _fc1_body(x_ref, w1_ref, b1_ref, gates_ref, gh_ref):
    e = pl.program_id(1)
    h = jnp.dot(x_ref[...], w1_ref[0], preferred_element_type=jnp.float32,
                precision=HIGH)
    h = jnp.maximum(h + b1_ref[0], 0.0)
    eidx = lax.broadcasted_iota(jnp.int32, (TB2, E), 1)
    gcol = jnp.sum(jnp.where(eidx == e, gates_ref[...], 0.0), axis=1,
                   keepdims=True)
    gh_ref[0, 0, 0, :] = jnp.sum(h * gcol, axis=0)


def _cv2(v):
    mean = jnp.mean(v)
    var1 = jnp.sum((v - mean) ** 2) / (E - 1)
    return var1 / (mean * mean + 1e-10)


def _final_body(gh_ref, w2_ref, b2_ref, imp_ref, load_ref, yt_ref, hw_ref,
                hb_ref, lng_ref, lnb_ref, scores_ref, aux_ref, pred_ref,
                acc_ref):
    e = pl.program_id(0)

    @pl.when(e == 0)
    def _():
        acc_ref[...] = jnp.zeros_like(acc_ref)

    gh = gh_ref[:, 0, 0, :]
    rows = jnp.reshape(gh, (B, NB2 // B, D)).sum(axis=1)
    acc_ref[0:B, :] += jnp.dot(rows, w2_ref[0],
                               preferred_element_type=jnp.float32,
                               precision=HIGH)

    @pl.when(e == E - 1)
    def _():
        imp_blk = imp_ref[:, 0, :]
        load_blk = load_ref[:, 0, :]
        imp_b = jnp.reshape(imp_blk, (B, NB1 // B, E)).sum(axis=1)
        importance = jnp.sum(imp_blk, axis=0)
        load = jnp.sum(load_blk, axis=0)
        aux = (_cv2(importance) + _cv2(load)) * 0.01
        aux_ref[...] = jnp.reshape(aux, (1, 1))

        mm = acc_ref[0:B, :] + jnp.dot(imp_b, b2_ref[...],
                                       preferred_element_type=jnp.float32,
                                       precision=HIGH)
        mu = jnp.mean(mm, axis=1, keepdims=True)
        var = jnp.mean((mm - mu) ** 2, axis=1, keepdims=True)
        fin = (mm - mu) * lax.rsqrt(var + 1e-5) * lng_ref[...] + lnb_ref[...]
        out = jnp.dot(fin, hw_ref[...], preferred_element_type=jnp.float32,
                      precision=HIGH) + hb_ref[...]
        scores = jax.nn.sigmoid(out)
        scores_ref[...] = scores
        pred_ref[...] = jnp.reshape(
            jnp.mean((scores - yt_ref[...]) ** 2), (1, 1))


def kernel(mm_embed, task_index, true_y, w_gate, fc1_w, fc1_b, fc2_w, fc2_b,
           head_w, head_b, ln_g, ln_b):
    x = mm_embed.reshape(T, D)

    gates, imp_blk, load_blk = pl.pallas_call(
        _gating_body,
        grid=(NB1,),
        in_specs=[
            pl.BlockSpec((TB1, D), lambda i: (i, 0)),
            pl.BlockSpec((D, E), lambda i: (0, 0)),
        ],
        out_specs=[
            pl.BlockSpec((TB1, E), lambda i: (i, 0)),
            pl.BlockSpec((1, 1, E), lambda i: (i, 0, 0)),
            pl.BlockSpec((1, 1, E), lambda i: (i, 0, 0)),
        ],
        out_shape=[
            jax.ShapeDtypeStruct((T, E), jnp.float32),
            jax.ShapeDtypeStruct((NB1, 1, E), jnp.float32),
            jax.ShapeDtypeStruct((NB1, 1, E), jnp.float32),
        ],
    )(x, w_gate)

    g_h = pl.pallas_call(
        _fc1_body,
        grid=(NB2, E),
        in_specs=[
            pl.BlockSpec((TB2, D), lambda i, e: (i, 0)),
            pl.BlockSpec((1, D, D), lambda i, e: (e, 0, 0)),
            pl.BlockSpec((1, 1, D), lambda i, e: (e, 0, 0)),
            pl.BlockSpec((TB2, E), lambda i, e: (i, 0)),
        ],
        out_specs=pl.BlockSpec((1, 1, 1, D), lambda i, e: (i, e, 0, 0)),
        out_shape=jax.ShapeDtypeStruct((NB2, E, 1, D), jnp.float32),
    )(x, fc1_w, fc1_b.reshape(E, 1, D), gates)

    scores, aux, pred = pl.pallas_call(
        _final_body,
        grid=(E,),
        in_specs=[
            pl.BlockSpec((NB2, 1, 1, D), lambda e: (0, e, 0, 0)),
            pl.BlockSpec((1, D, D), lambda e: (e, 0, 0)),
            pl.BlockSpec((E, D), lambda e: (0, 0)),
            pl.BlockSpec((NB1, 1, E), lambda e: (0, 0, 0)),
            pl.BlockSpec((NB1, 1, E), lambda e: (0, 0, 0)),
            pl.BlockSpec((B, 1), lambda e: (0, 0)),
            pl.BlockSpec((D, 1), lambda e: (0, 0)),
            pl.BlockSpec((1, 1), lambda e: (0, 0)),
            pl.BlockSpec((1, D), lambda e: (0, 0)),
            pl.BlockSpec((1, D), lambda e: (0, 0)),
        ],
        out_specs=[
            pl.BlockSpec((B, 1), lambda e: (0, 0)),
            pl.BlockSpec((1, 1), lambda e: (0, 0)),
            pl.BlockSpec((1, 1), lambda e: (0, 0)),
        ],
        out_shape=[
            jax.ShapeDtypeStruct((B, 1), jnp.float32),
            jax.ShapeDtypeStruct((1, 1), jnp.float32),
            jax.ShapeDtypeStruct((1, 1), jnp.float32),
        ],
        scratch_shapes=[pltpu.VMEM((8, D), jnp.float32)],
    )(g_h, fc2_w, fc2_b, imp_blk, load_blk, true_y,
      head_w, head_b.reshape(1, 1), ln_g.reshape(1, D), ln_b.reshape(1, D))

    return (scores, aux.reshape(()), pred.reshape(()))


# speedup vs baseline: 3.2778x; 3.2778x over previous
import jax
import jax.numpy as jnp
from jax import lax
from jax.experimental import pallas as pl
from jax.experimental.pallas import tpu as pltpu

B, M, D, E = 2, 2048, 768, 8
T = B * M
TB = 1024
NB = T // TB
HIGH = lax.Precision.HIGHEST


def _moe_body(xbf_ref, wg_ref, w1_ref, b1_ref, gh_ref, imp_ref, load_ref):
    xb = xbf_ref[...]
    logits = jnp.dot(xb, wg_ref[...], preferred_element_type=jnp.float32)
    eidx = lax.broadcasted_iota(jnp.int32, (TB, E), 1)
    m1 = jnp.max(logits, axis=1, keepdims=True)
    a1 = jnp.min(jnp.where(logits == m1, eidx, E), axis=1, keepdims=True)
    masked = jnp.where(eidx == a1, -jnp.inf, logits)
    m2 = jnp.max(masked, axis=1, keepdims=True)
    a2 = jnp.min(jnp.where(masked == m2, eidx, E), axis=1, keepdims=True)
    ed = jnp.exp(m2 - m1)
    g1 = 1.0 / (1.0 + ed)
    g2 = ed / (1.0 + ed)
    gates = (jnp.where(eidx == a1, g1, 0.0)
             + jnp.where(eidx == a2, g2, 0.0))
    imp_ref[0, 0, :] = jnp.sum(gates, axis=0)
    load_ref[0, 0, :] = jnp.sum((gates > 0.0).astype(jnp.float32), axis=0)
    for e in range(E):
        h = jnp.dot(xb, w1_ref[e], preferred_element_type=jnp.float32)
        h = jnp.maximum(h + b1_ref[e], 0.0)
        gh_ref[0, e, 0, :] = jnp.sum(h * gates[:, e:e + 1], axis=0)


def _cv2(v):
    mean = jnp.mean(v)
    var1 = jnp.sum((v - mean) ** 2) / (E - 1)
    return var1 / (mean * mean + 1e-10)


def _final_body(gh_ref, w2_ref, b2_ref, imp_ref, load_ref, yt_ref, hw_ref,
                hb_ref, lng_ref, lnb_ref, scores_ref, aux_ref, pred_ref,
                acc_ref):
    e = pl.program_id(0)

    @pl.when(e == 0)
    def _():
        acc_ref[...] = jnp.zeros_like(acc_ref)

    gh = gh_ref[:, 0, 0, :]
    rows = jnp.reshape(gh, (B, NB // B, D)).sum(axis=1)
    acc_ref[0:B, :] += jnp.dot(rows, w2_ref[0],
                               preferred_element_type=jnp.float32,
                               precision=HIGH)

    @pl.when(e == E - 1)
    def _():
        imp_blk = imp_ref[:, 0, :]
        load_blk = load_ref[:, 0, :]
        imp_b = jnp.reshape(imp_blk, (B, NB // B, E)).sum(axis=1)
        importance = jnp.sum(imp_blk, axis=0)
        load = jnp.sum(load_blk, axis=0)
        aux = (_cv2(importance) + _cv2(load)) * 0.01
        aux_ref[...] = jnp.reshape(aux, (1, 1))

        mm = acc_ref[0:B, :] + jnp.dot(imp_b, b2_ref[...],
                                       preferred_element_type=jnp.float32,
                                       precision=HIGH)
        mu = jnp.mean(mm, axis=1, keepdims=True)
        var = jnp.mean((mm - mu) ** 2, axis=1, keepdims=True)
        fin = (mm - mu) * lax.rsqrt(var + 1e-5) * lng_ref[...] + lnb_ref[...]
        out = jnp.dot(fin, hw_ref[...], preferred_element_type=jnp.float32,
                      precision=HIGH) + hb_ref[...]
        scores = jax.nn.sigmoid(out)
        scores_ref[...] = scores
        pred_ref[...] = jnp.reshape(
            jnp.mean((scores - yt_ref[...]) ** 2), (1, 1))


def kernel(mm_embed, task_index, true_y, w_gate, fc1_w, fc1_b, fc2_w, fc2_b,
           head_w, head_b, ln_g, ln_b):
    xbf = mm_embed.reshape(T, D).astype(jnp.bfloat16)
    w1bf = fc1_w.astype(jnp.bfloat16)

    g_h, imp_blk, load_blk = pl.pallas_call(
        _moe_body,
        grid=(NB,),
        in_specs=[
            pl.BlockSpec((TB, D), lambda i: (i, 0)),
            pl.BlockSpec((D, E), lambda i: (0, 0)),
            pl.BlockSpec((E, D, D), lambda i: (0, 0, 0)),
            pl.BlockSpec((E, 1, D), lambda i: (0, 0, 0)),
        ],
        out_specs=[
            pl.BlockSpec((1, E, 1, D), lambda i: (i, 0, 0, 0)),
            pl.BlockSpec((1, 1, E), lambda i: (i, 0, 0)),
            pl.BlockSpec((1, 1, E), lambda i: (i, 0, 0)),
        ],
        out_shape=[
            jax.ShapeDtypeStruct((NB, E, 1, D), jnp.float32),
            jax.ShapeDtypeStruct((NB, 1, E), jnp.float32),
            jax.ShapeDtypeStruct((NB, 1, E), jnp.float32),
        ],
    )(xbf, w_gate.astype(jnp.bfloat16), w1bf, fc1_b.reshape(E, 1, D))

    scores, aux, pred = pl.pallas_call(
        _final_body,
        grid=(E,),
        in_specs=[
            pl.BlockSpec((NB, 1, 1, D), lambda e: (0, e, 0, 0)),
            pl.BlockSpec((1, D, D), lambda e: (e, 0, 0)),
            pl.BlockSpec((E, D), lambda e: (0, 0)),
            pl.BlockSpec((NB, 1, E), lambda e: (0, 0, 0)),
            pl.BlockSpec((NB, 1, E), lambda e: (0, 0, 0)),
            pl.BlockSpec((B, 1), lambda e: (0, 0)),
            pl.BlockSpec((D, 1), lambda e: (0, 0)),
            pl.BlockSpec((1, 1), lambda e: (0, 0)),
            pl.BlockSpec((1, D), lambda e: (0, 0)),
            pl.BlockSpec((1, D), lambda e: (0, 0)),
        ],
        out_specs=[
            pl.BlockSpec((B, 1), lambda e: (0, 0)),
            pl.BlockSpec((1, 1), lambda e: (0, 0)),
            pl.BlockSpec((1, 1), lambda e: (0, 0)),
        ],
        out_shape=[
            jax.ShapeDtypeStruct((B, 1), jnp.float32),
            jax.ShapeDtypeStruct((1, 1), jnp.float32),
            jax.ShapeDtypeStruct((1, 1), jnp.float32),
        ],
        scratch_shapes=[pltpu.VMEM((8, D), jnp.float32)],
    )(g_h, fc2_w, fc2_b, imp_blk, load_blk, true_y,
      head_w, head_b.reshape(1, 1), ln_g.reshape(1, D), ln_b.reshape(1, D))

    return (scores, aux.reshape(()), pred.reshape(()))
